# baseline (speedup 1.0000x reference)
import functools

import jax
import jax.numpy as jnp
from jax import lax
from jax.experimental import pallas as pl
from jax.experimental.pallas import tpu as pltpu
from jax.experimental.pallas import tpu_sc as plsc


def _lane_broadcast(vec, j):
    idx = jnp.full((vec.shape[0], 1), j, jnp.int32)
    dnums = lax.GatherDimensionNumbers(
        offset_dims=(), collapsed_slice_dims=(0,), start_index_map=(0,))
    return lax.gather(vec, idx, dnums, slice_sizes=(1,),
                      mode=lax.GatherScatterMode.PROMISE_IN_BOUNDS)


N = 10000
NP = 10240
D = 128
NC = 2
NS = 16
LANES = 16
CHUNK = 80

NCH0 = 176
NCH1 = 80


def _mm_body(x_ref, w_ref, o_ref):
    o_ref[...] = jnp.dot(x_ref[...], w_ref[...],
                         preferred_element_type=jnp.float32)


def _fuse_body(p_ref, b_ref, w_ref, o_ref):
    h = jnp.maximum(p_ref[0, pl.ds(0, N)] + p_ref[1, pl.ds(0, N)]
                    + b_ref[...], 0.0)
    o_ref[...] = jnp.dot(h, w_ref[...], preferred_element_type=jnp.float32)


def _final_body(q_ref, b_ref, o_ref):
    o_ref[...] = jnp.maximum(q_ref[0, pl.ds(0, N)] + q_ref[1, pl.ds(0, N)]
                             + b_ref[...], 0.0)


def _tc_matmul(x, w):
    return pl.pallas_call(
        _mm_body,
        out_shape=jax.ShapeDtypeStruct((N, D), jnp.float32),
    )(x, w)


def _tc_fuse(p, b_row, w):
    return pl.pallas_call(
        _fuse_body,
        out_shape=jax.ShapeDtypeStruct((N, D), jnp.float32),
    )(p, b_row, w)


def _tc_final(q, b_row):
    return pl.pallas_call(
        _final_body,
        out_shape=jax.ShapeDtypeStruct((N, D), jnp.float32),
    )(q, b_row)


def _sc_body(g_hbm, sw_hbm, w_hbm, dst_hbm, out_hbm,
             agg_sh, pk0, pk1, pk2, pk3, wb0, wb1, wb2, wb3,
             db0, db1, db2, db3, db4, db5, db6, db7,
             rows0, rows1, rows2, rows3, gsem0, gsem1, isem0, isem1,
             ssem0, ssem1):
    cid = lax.axis_index("c")
    sid = lax.axis_index("s")
    rows_per_tile = NP // NS
    rows = (rows0, rows1, rows2, rows3)
    pk = (pk0, pk1, pk2, pk3)
    wv = (wb0, wb1, wb2, wb3)
    db = (db0, db1, db2, db3, db4, db5, db6, db7)
    gsem = (gsem0, gsem1)
    isem = (isem0, isem1)
    ssem = (ssem0, ssem1)

    nck = jnp.where(cid == 0, NCH0, NCH1)
    cbase = jnp.where(cid == 0, sid * NCH0, NS * NCH0 + sid * NCH1)

    def _off(c):
        return (cbase + c) * CHUNK

    def _idx_fetch_sync(c, slot4, slot8):
        pltpu.sync_copy(sw_hbm.at[pl.ds(_off(c), CHUNK)], pk[slot4])
        pltpu.sync_copy(w_hbm.at[pl.ds(_off(c), CHUNK)], wv[slot4])
        pltpu.sync_copy(dst_hbm.at[pl.ds(_off(c), CHUNK)], db[slot8])

    def _idx_fetch(c, slot4, slot8, sem):
        pltpu.async_copy(sw_hbm.at[pl.ds(_off(c), CHUNK)], pk[slot4], sem)
        pltpu.async_copy(w_hbm.at[pl.ds(_off(c), CHUNK)], wv[slot4], sem)
        pltpu.async_copy(dst_hbm.at[pl.ds(_off(c), CHUNK)], db[slot8], sem)

    def _idx_wait(c, slot4, slot8, sem):
        pltpu.make_async_copy(sw_hbm.at[pl.ds(_off(c), CHUNK)],
                              pk[slot4], sem).wait()
        pltpu.make_async_copy(w_hbm.at[pl.ds(_off(c), CHUNK)],
                              wv[slot4], sem).wait()
        pltpu.make_async_copy(dst_hbm.at[pl.ds(_off(c), CHUNK)],
                              db[slot8], sem).wait()

    def zbody(i, carry):
        for k in range(D // LANES):
            rows0[i, pl.ds(k * LANES, LANES)] = jnp.zeros((LANES,),
                                                          jnp.float32)
        return carry

    lax.fori_loop(0, CHUNK, zbody, 0)
    for t in range(rows_per_tile // CHUNK):
        pltpu.sync_copy(
            rows0,
            agg_sh.at[pl.ds(sid * rows_per_tile + t * CHUNK, CHUNK)])

    _idx_fetch_sync(0, 0, 0)
    _idx_fetch_sync(1, 1, 1)
    plsc.subcore_barrier()
    pltpu.async_copy(g_hbm.at[pk[0]], rows[0], gsem[0])
    pltpu.async_copy(g_hbm.at[pk[1]], rows[1], gsem[1])
    _idx_fetch(2, 2, 2, isem[0])
    _idx_fetch(3, 3, 3, isem[1])

    def oct_body(p, carry):
        for d in range(8):
            c = p * 8 + d
            r = d % 4
            buf = rows[r]
            pltpu.make_async_copy(g_hbm.at[pk[r]], buf, gsem[d % 2]).wait()

            @pl.when(c + 2 < nck)
            def _():
                _idx_wait(c + 2, (r + 2) % 4, (d + 2) % 8, isem[d % 2])
                pltpu.async_copy(g_hbm.at[pk[(r + 2) % 4]],
                                 rows[(r + 2) % 4], gsem[d % 2])

            def group_body(gi, carry2):
                w16 = wv[r][pl.ds(gi * LANES, LANES)]
                for j in range(LANES):
                    e = gi * LANES + j
                    wb = _lane_broadcast(w16, j)
                    for k in range(D // LANES):
                        sl = pl.ds(k * LANES, LANES)
                        buf[e, sl] = buf[e, sl] * wb
                return carry2

            lax.fori_loop(0, CHUNK // LANES, group_body, 0)

            pltpu.async_copy(buf, agg_sh.at[db[d]], ssem[d % 2], add=True)

            @pl.when(c >= 1)
            def _():
                pltpu.make_async_copy(rows[(r + 3) % 4],
                                      agg_sh.at[db[(d + 7) % 8]],
                                      ssem[(d + 1) % 2]).wait()

            @pl.when(c + 4 < nck)
            def _():
                _idx_fetch(c + 4, r, (d + 4) % 8, isem[d % 2])
        return carry

    lax.fori_loop(0, nck // 8, oct_body, 0)
    pltpu.make_async_copy(rows[3], agg_sh.at[db[7]], ssem[1]).wait()
    plsc.subcore_barrier()

    pltpu.sync_copy(agg_sh.at[pl.ds(sid * rows_per_tile, rows_per_tile)],
                    out_hbm.at[cid, pl.ds(sid * rows_per_tile, rows_per_tile)])


def _make_sc_pass():
    mesh = plsc.VectorSubcoreMesh(core_axis_name="c", subcore_axis_name="s")
    idx_i = pltpu.VMEM((CHUNK,), jnp.int32)
    idx_f = pltpu.VMEM((CHUNK,), jnp.float32)
    rbuf = pltpu.VMEM((CHUNK, D), jnp.float32)
    return pl.kernel(
        _sc_body,
        mesh=mesh,
        out_type=jax.ShapeDtypeStruct((NC, NP, D), jnp.float32),
        scratch_types=(
            [pltpu.VMEM_SHARED((NP, D), jnp.float32)]
            + [idx_i] * 4 + [idx_f] * 4 + [idx_i] * 8 + [rbuf] * 4
            + [pltpu.SemaphoreType.DMA] * 6
        ),
    )


def kernel(h, edge_index, edge_weight, W, b):
    E = edge_index.shape[1]
    epad = NS * (NCH0 + NCH1) * CHUNK
    assert epad >= E
    pad = epad - E

    src = edge_index[0]
    dst = edge_index[1]
    if pad:
        zpad_i = jnp.zeros((pad,), jnp.int32)
        src = jnp.concatenate([src, zpad_i])
        dst = jnp.concatenate([dst, zpad_i])
        edge_weight = jnp.concatenate([edge_weight,
                                       jnp.zeros((pad,), jnp.float32)])

    sc_pass = _make_sc_pass()

    g = _tc_matmul(h, W[0])
    p = sc_pass(g, src, edge_weight, dst)
    g2 = _tc_fuse(p, b[0].reshape(1, D), W[1])
    q = sc_pass(g2, src, edge_weight, dst)
    return _tc_final(q, b[1].reshape(1, D))


# speedup vs baseline: 2.0388x; 2.0388x over previous
import functools

import jax
import jax.numpy as jnp
from jax import lax
from jax.experimental import pallas as pl
from jax.experimental.pallas import tpu as pltpu
from jax.experimental.pallas import tpu_sc as plsc


def _lane_broadcast(vec, j):
    idx = jnp.full((vec.shape[0], 1), j, jnp.int32)
    dnums = lax.GatherDimensionNumbers(
        offset_dims=(), collapsed_slice_dims=(0,), start_index_map=(0,))
    return lax.gather(vec, idx, dnums, slice_sizes=(1,),
                      mode=lax.GatherScatterMode.PROMISE_IN_BOUNDS)


N = 10000
NP = 10240
D = 128
NC = 2
NS = 16
LANES = 16
CHUNK = 112

NCH0 = 129
NCH1 = 51


def _mm_body(x_ref, w_ref, o_ref):
    o_ref[...] = jnp.dot(x_ref[...], w_ref[...],
                         preferred_element_type=jnp.float32)


def _fuse_body(p_ref, b_ref, w_ref, o_ref):
    h = jnp.maximum(p_ref[0, pl.ds(0, N)] + p_ref[1, pl.ds(0, N)]
                    + b_ref[...], 0.0)
    o_ref[...] = jnp.dot(h, w_ref[...], preferred_element_type=jnp.float32)


def _final_body(q_ref, b_ref, o_ref):
    o_ref[...] = jnp.maximum(q_ref[0, pl.ds(0, N)] + q_ref[1, pl.ds(0, N)]
                             + b_ref[...], 0.0)


def _tc_matmul(x, w):
    return pl.pallas_call(
        _mm_body,
        out_shape=jax.ShapeDtypeStruct((N, D), jnp.float32),
    )(x, w)


def _tc_fuse(p, b_row, w):
    return pl.pallas_call(
        _fuse_body,
        out_shape=jax.ShapeDtypeStruct((N, D), jnp.float32),
    )(p, b_row, w)


def _tc_final(q, b_row):
    return pl.pallas_call(
        _final_body,
        out_shape=jax.ShapeDtypeStruct((N, D), jnp.float32),
    )(q, b_row)


def _sc_body(g_hbm, sw_hbm, w_hbm, dst_hbm, out_hbm,
             agg_sh, pk0, pk1, pk2, wb0, wb1, wb2, db0, db1, db2,
             rows0, rows1, rows2, gsem, isem, ssem0, ssem1, ssem2):
    cid = lax.axis_index("c")
    sid = lax.axis_index("s")
    rows_per_tile = NP // NS
    rows = (rows0, rows1, rows2)
    pk = (pk0, pk1, pk2)
    wv = (wb0, wb1, wb2)
    db = (db0, db1, db2)
    ssem = (ssem0, ssem1, ssem2)

    nck = jnp.where(cid == 0, NCH0, NCH1)
    cbase = jnp.where(cid == 0, sid * NCH0, NS * NCH0 + sid * NCH1)

    def _off(c):
        return (cbase + c) * CHUNK

    ZR = 80
    def zbody(i, carry):
        for k in range(D // LANES):
            rows0[i, pl.ds(k * LANES, LANES)] = jnp.zeros((LANES,),
                                                          jnp.float32)
        return carry

    lax.fori_loop(0, ZR, zbody, 0)
    for t in range(rows_per_tile // ZR):
        pltpu.sync_copy(
            rows0.at[pl.ds(0, ZR)],
            agg_sh.at[pl.ds(sid * rows_per_tile + t * ZR, ZR)])

    pltpu.sync_copy(sw_hbm.at[pl.ds(_off(0), CHUNK)], pk[0])
    pltpu.sync_copy(w_hbm.at[pl.ds(_off(0), CHUNK)], wv[0])
    pltpu.sync_copy(dst_hbm.at[pl.ds(_off(0), CHUNK)], db[0])
    plsc.subcore_barrier()
    pltpu.async_copy(g_hbm.at[pk[0]], rows[0], gsem)
    pltpu.async_copy(sw_hbm.at[pl.ds(_off(1), CHUNK)], pk[1], isem)
    pltpu.async_copy(w_hbm.at[pl.ds(_off(1), CHUNK)], wv[1], isem)
    pltpu.async_copy(dst_hbm.at[pl.ds(_off(1), CHUNK)], db[1], isem)

    def tri_body(p, carry):
        for r in range(3):
            c = p * 3 + r
            buf = rows[r]
            pltpu.make_async_copy(g_hbm.at[pk[r]], buf, gsem).wait()

            @pl.when(c + 1 < nck)
            def _():
                r1 = (r + 1) % 3
                pltpu.make_async_copy(
                    sw_hbm.at[pl.ds(_off(c + 1), CHUNK)],
                    pk[r1], isem).wait()
                pltpu.make_async_copy(
                    w_hbm.at[pl.ds(_off(c + 1), CHUNK)],
                    wv[r1], isem).wait()
                pltpu.make_async_copy(
                    dst_hbm.at[pl.ds(_off(c + 1), CHUNK)],
                    db[r1], isem).wait()
                pltpu.async_copy(g_hbm.at[pk[r1]], rows[r1], gsem)

            def group_body(gi, carry2):
                w16 = wv[r][pl.ds(gi * LANES, LANES)]
                for j in range(LANES):
                    e = gi * LANES + j
                    wb = _lane_broadcast(w16, j)
                    for k in range(D // LANES):
                        sl = pl.ds(k * LANES, LANES)
                        buf[e, sl] = buf[e, sl] * wb
                return carry2

            lax.fori_loop(0, CHUNK // LANES, group_body, 0)

            pltpu.async_copy(buf, agg_sh.at[db[r]], ssem[r], add=True)

            @pl.when(c >= 1)
            def _():
                r2 = (r + 2) % 3
                pltpu.make_async_copy(rows[r2], agg_sh.at[db[r2]],
                                      ssem[r2]).wait()

            @pl.when(c + 2 < nck)
            def _():
                r2 = (r + 2) % 3
                pltpu.async_copy(sw_hbm.at[pl.ds(_off(c + 2), CHUNK)],
                                 pk[r2], isem)
                pltpu.async_copy(w_hbm.at[pl.ds(_off(c + 2), CHUNK)],
                                 wv[r2], isem)
                pltpu.async_copy(dst_hbm.at[pl.ds(_off(c + 2), CHUNK)],
                                 db[r2], isem)
        return carry

    lax.fori_loop(0, nck // 3, tri_body, 0)
    pltpu.make_async_copy(rows[2], agg_sh.at[db[2]], ssem[2]).wait()
    plsc.subcore_barrier()

    pltpu.sync_copy(agg_sh.at[pl.ds(sid * rows_per_tile, rows_per_tile)],
                    out_hbm.at[cid, pl.ds(sid * rows_per_tile, rows_per_tile)])


def _make_sc_pass():
    mesh = plsc.VectorSubcoreMesh(core_axis_name="c", subcore_axis_name="s")
    return pl.kernel(
        _sc_body,
        mesh=mesh,
        out_type=jax.ShapeDtypeStruct((NC, NP, D), jnp.float32),
        scratch_types=[
            pltpu.VMEM_SHARED((NP, D), jnp.float32),
            pltpu.VMEM((CHUNK,), jnp.int32),
            pltpu.VMEM((CHUNK,), jnp.int32),
            pltpu.VMEM((CHUNK,), jnp.int32),
            pltpu.VMEM((CHUNK,), jnp.float32),
            pltpu.VMEM((CHUNK,), jnp.float32),
            pltpu.VMEM((CHUNK,), jnp.float32),
            pltpu.VMEM((CHUNK,), jnp.int32),
            pltpu.VMEM((CHUNK,), jnp.int32),
            pltpu.VMEM((CHUNK,), jnp.int32),
            pltpu.VMEM((CHUNK, D), jnp.float32),
            pltpu.VMEM((CHUNK, D), jnp.float32),
            pltpu.VMEM((CHUNK, D), jnp.float32),
            pltpu.SemaphoreType.DMA,
            pltpu.SemaphoreType.DMA,
            pltpu.SemaphoreType.DMA,
            pltpu.SemaphoreType.DMA,
            pltpu.SemaphoreType.DMA,
        ],
    )


def kernel(h, edge_index, edge_weight, W, b):
    E = edge_index.shape[1]
    epad = NS * (NCH0 + NCH1) * CHUNK
    assert epad >= E
    pad = epad - E

    src = edge_index[0]
    dst = edge_index[1]
    if pad:
        zpad_i = jnp.zeros((pad,), jnp.int32)
        src = jnp.concatenate([src, zpad_i])
        dst = jnp.concatenate([dst, zpad_i])
        edge_weight = jnp.concatenate([edge_weight,
                                       jnp.zeros((pad,), jnp.float32)])

    sc_pass = _make_sc_pass()

    g = _tc_matmul(h, W[0])
    p = sc_pass(g, src, edge_weight, dst)
    g2 = _tc_fuse(p, b[0].reshape(1, D), W[1])
    q = sc_pass(g2, src, edge_weight, dst)
    return _tc_final(q, b[1].reshape(1, D))
